# baseline (speedup 1.0000x reference)
import functools

import jax
import jax.numpy as jnp
from jax import lax
from jax.experimental import pallas as pl
from jax.experimental.pallas import tpu as pltpu
from jax.experimental.pallas import tpu_sc as plsc

_B, _S, _D, _E = 4, 8192, 768, 64
_NW = 32
_SC_ROWS_PB = 1536
_TC_ROWS_PB = _S - _SC_ROWS_PB
_WPB = _NW // _B
_RPW = _SC_ROWS_PB // _WPB
_RCHUNK = 24
_NITER = _RPW // _RCHUNK
_NV = _D // 16
_NBUF = 4
_CHUNK_TC = 4096

_mesh = plsc.VectorSubcoreMesh(
    core_axis_name="c", subcore_axis_name="s", num_cores=2, num_subcores=16
)


@functools.partial(
    pl.kernel,
    out_type=jax.ShapeDtypeStruct((_NW, _D), jnp.float32),
    mesh=_mesh,
    scratch_types=[
        pltpu.VMEM((_NBUF, _RCHUNK * _D), jnp.float32),
        pltpu.VMEM((_D,), jnp.float32),
        pltpu.SemaphoreType.DMA,
        pltpu.SemaphoreType.DMA,
        pltpu.SemaphoreType.DMA,
        pltpu.SemaphoreType.DMA,
    ],
)
def _sc_mean(x_hbm, out_hbm, buf, accv, sem0, sem1, sem2, sem3):
    wid = lax.axis_index("s") * 2 + lax.axis_index("c")
    b = wid // _WPB
    k_in_b = wid % _WPB
    base = (b * _S + _TC_ROWS_PB + k_in_b * _RPW) * _D
    sems = (sem0, sem1, sem2, sem3)

    def dma(i, k):
        return pltpu.make_async_copy(
            x_hbm.at[pl.ds(base + i * (_RCHUNK * _D), _RCHUNK * _D)],
            buf.at[k],
            sems[k],
        )

    zero = jnp.zeros((16,), jnp.float32)
    for j in range(_NV):
        accv[pl.ds(j * 16, 16)] = zero

    zerov = jnp.zeros((16,), jnp.float32)

    def process(k):
        def jbody(j, c, k=k):
            off0 = j * 16
            accs = [accv[pl.ds(off0, 16)], zerov, zerov, zerov]
            for r in range(_RCHUNK):
                accs[r % 4] = accs[r % 4] + buf[k, pl.ds(off0 + r * _D, 16)]
            accv[pl.ds(off0, 16)] = (accs[0] + accs[1]) + (accs[2] + accs[3])
            return c

        lax.fori_loop(0, _NV, jbody, 0)

    for p in range(_NBUF - 1):
        dma(p, p).start()

    def outer(i4, carry):
        i = i4 * _NBUF
        for p in range(_NBUF):
            dma(i + p, p).wait()
            process(p)

            @pl.when(i + p + _NBUF - 1 < _NITER)
            def _(i=i, p=p):
                dma(i + p + _NBUF - 1, (p + _NBUF - 1) % _NBUF).start()

        return carry

    lax.fori_loop(0, _NITER // _NBUF, outer, 0)
    pltpu.sync_copy(accv, out_hbm.at[wid])


def _gate_tail(m, wh, bh, wo, bo):
    h = jnp.dot(m, wh, preferred_element_type=jnp.float32) + bh
    h = h * jax.nn.sigmoid(h)
    logits = jnp.dot(h, wo, preferred_element_type=jnp.float32) + bo
    iota = lax.broadcasted_iota(jnp.int32, logits.shape, 1)
    v1 = jnp.max(logits, axis=1, keepdims=True)
    i1 = jnp.min(jnp.where(logits == v1, iota, _E), axis=1, keepdims=True)
    masked = jnp.where(iota == i1, -jnp.inf, logits)
    v2 = jnp.max(masked, axis=1, keepdims=True)
    i2 = jnp.min(jnp.where(masked == v2, iota, _E), axis=1, keepdims=True)
    e2 = jnp.exp(v2 - v1)
    denom = 1.0 + e2
    w = jnp.concatenate([1.0 / denom, e2 / denom], axis=1)
    i = jnp.concatenate([i1, i2], axis=1)
    return w, i


def _tc_reduce_body(x_ref, o_ref):
    b = pl.program_id(0)
    c = pl.program_id(1)
    partial = jnp.sum(x_ref[0], axis=0, keepdims=True)

    @pl.when(c == 0)
    def _():
        o_ref[pl.ds(b, 1), :] = partial

    @pl.when(c > 0)
    def _():
        o_ref[pl.ds(b, 1), :] += partial


def _tail_body(a_ref, wh_ref, bh_ref, wo_ref, bo_ref, w_ref, i_ref):
    m = a_ref[...] * (1.0 / _S)
    w, i = _gate_tail(m, wh_ref[...], bh_ref[...], wo_ref[...], bo_ref[...])
    w_ref[...] = w
    i_ref[...] = i


def kernel(x, W_hidden, b_hidden, W_out, b_out):
    acc = pl.pallas_call(
        _tc_reduce_body,
        grid=(_B, _S // _CHUNK_TC),
        in_specs=[pl.BlockSpec((1, _CHUNK_TC, _D), lambda b, c: (b, c, 0))],
        out_specs=pl.BlockSpec((_B, _D), lambda b, c: (0, 0)),
        out_shape=jax.ShapeDtypeStruct((_B, _D), jnp.float32),
    )(x)
    bh = b_hidden.reshape(1, _D)
    bo = b_out.reshape(1, _E)
    w, i = pl.pallas_call(
        _tail_body,
        out_shape=[
            jax.ShapeDtypeStruct((_B, 2), jnp.float32),
            jax.ShapeDtypeStruct((_B, 2), jnp.int32),
        ],
    )(acc, W_hidden, bh, W_out, bo)
    return w, i


# speedup vs baseline: 1.0270x; 1.0270x over previous
import jax
import jax.numpy as jnp
from jax import lax
from jax.experimental import pallas as pl
from jax.experimental.pallas import tpu as pltpu

_B, _S, _D, _E = 4, 8192, 768, 64
_CHUNK = 2048
_NC = _S // _CHUNK


def _gate_tail(m, wh, bh, wo, bo):
    h = jnp.dot(m, wh, preferred_element_type=jnp.float32) + bh
    h = h * jax.nn.sigmoid(h)
    logits = jnp.dot(h, wo, preferred_element_type=jnp.float32) + bo
    iota = lax.broadcasted_iota(jnp.int32, logits.shape, 1)
    v1 = jnp.max(logits, axis=1, keepdims=True)
    i1 = jnp.min(jnp.where(logits == v1, iota, _E), axis=1, keepdims=True)
    masked = jnp.where(iota == i1, -jnp.inf, logits)
    v2 = jnp.max(masked, axis=1, keepdims=True)
    i2 = jnp.min(jnp.where(masked == v2, iota, _E), axis=1, keepdims=True)
    e2 = jnp.exp(v2 - v1)
    denom = 1.0 + e2
    w = jnp.concatenate([1.0 / denom, e2 / denom], axis=1)
    i = jnp.concatenate([i1, i2], axis=1)
    return w, i


def _tc_body(x_ref, wh_ref, bh_ref, wo_ref, bo_ref, w_ref, i_ref, acc_ref):
    b = pl.program_id(0)
    c = pl.program_id(1)
    partial = jnp.sum(x_ref[0], axis=0, keepdims=True)

    @pl.when(c == 0)
    def _():
        acc_ref[pl.ds(b, 1), :] = partial

    @pl.when(c > 0)
    def _():
        acc_ref[pl.ds(b, 1), :] += partial

    @pl.when((b == _B - 1) & (c == _NC - 1))
    def _():
        m = acc_ref[...] * (1.0 / _S)
        w, i = _gate_tail(m, wh_ref[...], bh_ref[...], wo_ref[...], bo_ref[...])
        w_ref[...] = w
        i_ref[...] = i


def kernel(x, W_hidden, b_hidden, W_out, b_out):
    bh = b_hidden.reshape(1, _D)
    bo = b_out.reshape(1, _E)
    w, i = pl.pallas_call(
        _tc_body,
        grid=(_B, _NC),
        in_specs=[
            pl.BlockSpec((1, _CHUNK, _D), lambda b, c: (b, c, 0)),
            pl.BlockSpec((_D, _D), lambda b, c: (0, 0)),
            pl.BlockSpec((1, _D), lambda b, c: (0, 0)),
            pl.BlockSpec((_D, _E), lambda b, c: (0, 0)),
            pl.BlockSpec((1, _E), lambda b, c: (0, 0)),
        ],
        out_specs=[
            pl.BlockSpec((_B, 2), lambda b, c: (0, 0)),
            pl.BlockSpec((_B, 2), lambda b, c: (0, 0)),
        ],
        out_shape=[
            jax.ShapeDtypeStruct((_B, 2), jnp.float32),
            jax.ShapeDtypeStruct((_B, 2), jnp.int32),
        ],
        scratch_shapes=[pltpu.VMEM((_B, _D), jnp.float32)],
    )(x, W_hidden, bh, W_out, bo)
    return w, i
